# baseline (speedup 1.0000x reference)
import functools

import jax
import jax.numpy as jnp
from jax import lax
from jax.experimental import pallas as pl
from jax.experimental.pallas import tpu as pltpu
from jax.experimental.pallas import tpu_sc as plsc


def _pad_body(in_ref, out_ref):
    x = in_ref[...]
    out_ref[...] = jnp.concatenate((x, x), axis=1)


@functools.lru_cache(maxsize=None)
def _make_pad(V, D, Rb):
    assert V % Rb == 0
    return pl.pallas_call(
        _pad_body,
        grid=(V // Rb,),
        in_specs=[pl.BlockSpec((Rb, D), lambda g: (g, 0))],
        out_specs=pl.BlockSpec((Rb, 2 * D), lambda g: (g, 0)),
        out_shape=jax.ShapeDtypeStruct((V, 2 * D), jnp.float32),
    )


@functools.lru_cache(maxsize=None)
def _make_gather(V, D, B, C):
    info = plsc.get_sparse_core_info()
    NC, NS, L = info.num_cores, info.num_subcores, info.num_lanes
    NW = NC * NS
    assert B % (NW * C * 2) == 0 and D % L == 0
    b_per_w = B // NW
    n_chunks = b_per_w // C
    nvec = D // L

    mesh = plsc.VectorSubcoreMesh(core_axis_name="c", subcore_axis_name="s")

    @functools.partial(
        pl.kernel,
        mesh=mesh,
        out_type=jax.ShapeDtypeStruct((B, D), jnp.float32),
        scratch_types=[
            pltpu.VMEM((C,), jnp.int32),
            pltpu.VMEM((C,), jnp.int32),
            pltpu.VMEM((C, 2 * D), jnp.float32),
            pltpu.VMEM((C, 2 * D), jnp.float32),
            pltpu.VMEM((C, D), jnp.float32),
            pltpu.VMEM((C, D), jnp.float32),
            pltpu.SemaphoreType.DMA((2,)),
            pltpu.SemaphoreType.DMA((2,)),
        ],
    )
    def k(pad_hbm, idx_hbm, out_hbm, i0, i1, r0, r1, c0, c1, gsem, ssem):
        idx_v = [i0, i1]
        rows_v = [r0, r1]
        cv = [c0, c1]
        wid = lax.axis_index("s") * NC + lax.axis_index("c")
        base = wid * b_per_w

        def start_gather(i, p):
            pltpu.sync_copy(idx_hbm.at[pl.ds(base + i * C, C)], idx_v[p])
            pltpu.async_copy(pad_hbm.at[idx_v[p]], rows_v[p], gsem.at[p])

        def wait_gather(p):
            pltpu.make_async_copy(
                pad_hbm.at[idx_v[p]], rows_v[p], gsem.at[p]
            ).wait()

        def start_store(i, p):
            pltpu.async_copy(
                cv[p], out_hbm.at[pl.ds(base + i * C, C)], ssem.at[p]
            )

        def wait_store(i, p):
            pltpu.make_async_copy(
                cv[p], out_hbm.at[pl.ds(base + i * C, C)], ssem.at[p]
            ).wait()

        def compact(p):
            def body(c, carry):
                for u in range(2):
                    for l in range(nvec):
                        cv[p][2 * c + u, pl.ds(l * L, L)] = rows_v[p][
                            2 * c + u, pl.ds(l * L, L)
                        ]
                return carry

            lax.fori_loop(0, C // 2, body, 0)

        start_gather(0, 0)

        def body(j, carry):
            for p in range(2):
                i = j * 2 + p

                @pl.when(i + 1 < n_chunks)
                def _prefetch():
                    start_gather(i + 1, 1 - p)

                wait_gather(p)

                @pl.when(i >= 2)
                def _drain():
                    wait_store(i - 2, p)

                compact(p)
                start_store(i, p)
            return carry

        lax.fori_loop(0, n_chunks // 2, body, 0)
        for i in range(n_chunks - 2, n_chunks):
            wait_store(i, i % 2)

    return k


def kernel(x, table):
    BATCH, HIST = x.shape
    V, D = table.shape
    B = BATCH * HIST
    xf = x.reshape(B).astype(jnp.int32)
    tablepad = _make_pad(V, D, 2000)(table)
    out = _make_gather(V, D, B, 200)(tablepad, xf)
    return out.reshape(BATCH, HIST, D)


# speedup vs baseline: 1.2235x; 1.2235x over previous
import functools
import jax
import jax.numpy as jnp
from jax import lax
from jax.experimental import pallas as pl
from jax.experimental.pallas import tpu as pltpu
from jax.experimental.pallas import tpu_sc as plsc


def _tpack_body(in_ref, out_ref):
    t = in_ref[...]
    tt = t.T
    h = tt.shape[0] // 2
    out_ref[...] = jnp.concatenate((tt[:h], tt[h:]), axis=1)


@functools.lru_cache(maxsize=None)
def _make_tpack(V, D, Bi):
    grid = -(-V // Bi)
    return pl.pallas_call(
        _tpack_body,
        grid=(grid,),
        in_specs=[pl.BlockSpec((D, Bi), lambda g: (0, g))],
        out_specs=pl.BlockSpec((Bi // 2, 2 * D), lambda g: (g, 0)),
        out_shape=jax.ShapeDtypeStruct((grid * Bi // 2, 2 * D), jnp.float32),
    )


@functools.lru_cache(maxsize=None)
def _make_gather(V, D, BATCH, HIST, C, nbuf=4, lead=2):
    info = plsc.get_sparse_core_info()
    NC, NS = info.num_cores, info.num_subcores
    NW = NC * NS
    assert BATCH % NW == 0 and HIST == C
    bat_per_w = BATCH // NW
    n_chunks = bat_per_w

    mesh = plsc.VectorSubcoreMesh(core_axis_name="c", subcore_axis_name="s")

    @functools.partial(
        pl.kernel,
        mesh=mesh,
        out_type=jax.ShapeDtypeStruct((BATCH, HIST, D), jnp.float32),
        scratch_types=[
            pltpu.VMEM((nbuf, C), jnp.int32),
            pltpu.VMEM((nbuf, C, D), jnp.float32),
            pltpu.SemaphoreType.DMA((nbuf,)),
            pltpu.SemaphoreType.DMA((nbuf,)),
        ],
        compiler_params=pltpu.CompilerParams(use_tc_tiling_on_sc=False),
    )
    def k(table_hbm, idx_hbm, out_hbm, idx_v, rows_v, gsem, ssem):
        wid = lax.axis_index("s") * NC + lax.axis_index("c")
        base = wid * bat_per_w

        def start_gather(i, p):
            pltpu.sync_copy(idx_hbm.at[pl.ds((base + i) * C, C)], idx_v.at[p])
            pltpu.async_copy(table_hbm.at[idx_v.at[p]], rows_v.at[p], gsem.at[p])

        def wait_gather(i, p):
            pltpu.make_async_copy(
                table_hbm.at[idx_v.at[p]], rows_v.at[p], gsem.at[p]
            ).wait()

        def start_store(i, p):
            pltpu.async_copy(rows_v.at[p], out_hbm.at[base + i], ssem.at[p])

        def wait_store(i, p):
            pltpu.make_async_copy(
                rows_v.at[p], out_hbm.at[base + i], ssem.at[p]
            ).wait()

        for p in range(lead):
            start_gather(p, p)

        def body(j, carry):
            for p in range(nbuf):
                i = j * nbuf + p
                wait_gather(i, p)
                start_store(i, p)
                q = (p + lead) % nbuf

                @pl.when(i + lead < n_chunks)
                def _issue():
                    @pl.when(i >= lead)
                    def _drain():
                        wait_store(i - lead, q)

                    start_gather(i + lead, q)

            return carry

        lax.fori_loop(0, n_chunks // nbuf, body, 0)
        for i in range(n_chunks - lead, n_chunks):
            wait_store(i, i % nbuf)

    return k


def kernel(x, table):
    BATCH, HIST = x.shape
    V, D = table.shape
    B = BATCH * HIST
    Bi = 2048
    Hb = Bi // 2
    xf = x.reshape(B).astype(jnp.int32)
    g = xf // Bi
    k = xf % Bi
    xperm = g * Bi + jnp.where(k < Hb, 2 * k, 2 * (k - Hb) + 1)
    tpack = _make_tpack(V, D, Bi)(table.T)
    Vp = 2 * tpack.shape[0]
    tlin = tpack.reshape(Vp, D)
    out = _make_gather(Vp, D, BATCH, HIST, HIST)(tlin, xperm)
    return out
